# Initial kernel scaffold; baseline (speedup 1.0000x reference)
import jax
import jax.numpy as jnp
from jax.experimental import pallas as pl


def kernel(x, edge_index, h_prev, W_in, b_in, Wl1, bl1, Wr1, Wl2, bl2, Wr2, g1, be1, g2, be2, Wjk, bjk, Wih, Whh, bih, bhh, Wc1, bc1, gc, bec, Wc2, bc2, Wc3, bc3):
    raise NotImplementedError("write your pallas kernel here")


# speedup vs baseline: 4.7003x; 4.7003x over previous
import functools
import math

import jax
import jax.numpy as jnp
from jax import lax
from jax.experimental import pallas as pl
from jax.experimental.pallas import tpu as pltpu
from jax.experimental.pallas import tpu_sc as plsc

_N = 10000
_E = 320000
_DF = 128
_H = 64
_W = 128

_NUM_SC = 2
_TILES = 16
_WORKERS = _NUM_SC * _TILES
_EW = _E // _WORKERS
_CHUNK = 80
_NCHUNK = _EW // _CHUNK
_NPAD = 10240
_RPT = _NPAD // _TILES
_ZROWS = 128

_BN_INV = float(1.0 / math.sqrt(1.0 + 1e-5))


def _sc_agg(h, src, dst):
    @functools.partial(
        pl.kernel,
        out_type=jax.ShapeDtypeStruct((_NUM_SC, _NPAD, _W), jnp.float32),
        mesh=plsc.VectorSubcoreMesh(core_axis_name="c", subcore_axis_name="s"),
        scratch_types=[
            pltpu.VMEM((_CHUNK,), jnp.int32),
            pltpu.VMEM((_CHUNK,), jnp.int32),
            pltpu.VMEM((_CHUNK, _W), jnp.float32),
            pltpu.VMEM((_ZROWS, _W), jnp.float32),
            pltpu.VMEM_SHARED((_NPAD, _W), jnp.float32),
            pltpu.SemaphoreType.DMA,
        ],
    )
    def k(h_hbm, src_hbm, dst_hbm, out_hbm, src_v, dst_v, rows_v, zb_v, agg_s, sem):
        cid = lax.axis_index("c")
        sid = lax.axis_index("s")
        wid = cid * _TILES + sid

        @pl.loop(0, _ZROWS)
        def _(i):
            @pl.loop(0, _W // 16)
            def _(j):
                zb_v[i, pl.ds(j * 16, 16)] = jnp.zeros((16,), jnp.float32)

        @pl.loop(0, _RPT // _ZROWS)
        def _(kk):
            pltpu.sync_copy(zb_v, agg_s.at[pl.ds(sid * _RPT + kk * _ZROWS, _ZROWS)])

        plsc.subcore_barrier()

        @pl.loop(0, _NCHUNK)
        def _(j):
            e0 = wid * _EW + j * _CHUNK
            pltpu.sync_copy(src_hbm.at[pl.ds(e0, _CHUNK)], src_v)
            pltpu.sync_copy(dst_hbm.at[pl.ds(e0, _CHUNK)], dst_v)
            pltpu.async_copy(h_hbm.at[src_v], rows_v, sem).wait()
            pltpu.sync_copy(rows_v, agg_s.at[dst_v], add=True)

        plsc.subcore_barrier()
        pltpu.sync_copy(
            agg_s.at[pl.ds(sid * _RPT, _RPT)],
            out_hbm.at[cid, pl.ds(sid * _RPT, _RPT)],
        )

    return k(h, src, dst)


_ROWS = 2000


def _tag_tail(r):
    return (lax.broadcasted_iota(jnp.int32, (r, _W - _H), 1) == 0).astype(
        jnp.float32
    )


def _tc_input(x, w_t, b):
    def body(x_ref, w_ref, b_ref, o_ref):
        y = jnp.dot(x_ref[...], w_ref[...], preferred_element_type=jnp.float32)
        o_ref[:, : _H] = jnp.maximum(y + b_ref[...], 0.0)
        o_ref[:, _H :] = _tag_tail(_ROWS)

    return pl.pallas_call(
        body,
        grid=(_N // _ROWS,),
        in_specs=[
            pl.BlockSpec((_ROWS, _DF), lambda i: (i, 0)),
            pl.BlockSpec((_DF, _H), lambda i: (0, 0)),
            pl.BlockSpec((1, _H), lambda i: (0, 0)),
        ],
        out_specs=pl.BlockSpec((_ROWS, _W), lambda i: (i, 0)),
        out_shape=jax.ShapeDtypeStruct((_N, _W), jnp.float32),
    )(x, w_t, b)


def _tc_sage(aggp, h, wl_t, bl, wr_t, g, be):
    def body(a_ref, h_ref, wl_ref, bl_ref, wr_ref, g_ref, be_ref, o_ref):
        p = a_ref[0] + a_ref[1]
        cnt = p[:, _H : _H + 1]
        mean = p[:, : _H] / jnp.maximum(cnt, 1.0)
        hh = h_ref[:, : _H]
        y = (
            jnp.dot(mean, wl_ref[...], preferred_element_type=jnp.float32)
            + bl_ref[...]
            + jnp.dot(hh, wr_ref[...], preferred_element_type=jnp.float32)
        )
        y = g_ref[...] * (y * _BN_INV) + be_ref[...]
        o_ref[:, : _H] = jnp.maximum(y, 0.0) + hh
        o_ref[:, _H :] = _tag_tail(_ROWS)

    return pl.pallas_call(
        body,
        grid=(_N // _ROWS,),
        in_specs=[
            pl.BlockSpec((_NUM_SC, _ROWS, _W), lambda i: (0, i, 0)),
            pl.BlockSpec((_ROWS, _W), lambda i: (i, 0)),
            pl.BlockSpec((_H, _H), lambda i: (0, 0)),
            pl.BlockSpec((1, _H), lambda i: (0, 0)),
            pl.BlockSpec((_H, _H), lambda i: (0, 0)),
            pl.BlockSpec((1, _H), lambda i: (0, 0)),
            pl.BlockSpec((1, _H), lambda i: (0, 0)),
        ],
        out_specs=pl.BlockSpec((_ROWS, _W), lambda i: (i, 0)),
        out_shape=jax.ShapeDtypeStruct((_N, _W), jnp.float32),
    )(aggp, h, wl_t, bl, wr_t, g, be)


def _sigmoid(x):
    return 1.0 / (1.0 + jnp.exp(-x))


def _tc_head(h0, h1, h2, hp, wjk_t, bjk, wih_t, bih, whh_t, bhh,
             wc1_t, bc1, gc, bec, wc2_t, bc2, wc3_t, bc3):
    def body(h0_ref, h1_ref, h2_ref, hp_ref, wjk_ref, bjk_ref, wih_ref, bih_ref,
             whh_ref, bhh_ref, wc1_ref, bc1_ref, gc_ref, bec_ref, wc2_ref,
             bc2_ref, wc3_ref, bc3_ref, oh_ref, ol_ref):
        hj = (
            jnp.dot(h0_ref[:, : _H], wjk_ref[0], preferred_element_type=jnp.float32)
            + jnp.dot(h1_ref[:, : _H], wjk_ref[1], preferred_element_type=jnp.float32)
            + jnp.dot(h2_ref[:, : _H], wjk_ref[2], preferred_element_type=jnp.float32)
        )
        h = jnp.maximum(hj + bjk_ref[...], 0.0)
        hprev = hp_ref[...]
        ir = jnp.dot(h, wih_ref[0], preferred_element_type=jnp.float32) + bih_ref[0]
        iz = jnp.dot(h, wih_ref[1], preferred_element_type=jnp.float32) + bih_ref[1]
        inn = jnp.dot(h, wih_ref[2], preferred_element_type=jnp.float32) + bih_ref[2]
        hr = jnp.dot(hprev, whh_ref[0], preferred_element_type=jnp.float32) + bhh_ref[0]
        hz = jnp.dot(hprev, whh_ref[1], preferred_element_type=jnp.float32) + bhh_ref[1]
        hn = jnp.dot(hprev, whh_ref[2], preferred_element_type=jnp.float32) + bhh_ref[2]
        r = _sigmoid(ir + hr)
        z = _sigmoid(iz + hz)
        n = jnp.tanh(inn + r * hn)
        hc = (1.0 - z) * n + z * hprev
        oh_ref[...] = hc
        c = jnp.dot(hc, wc1_ref[...], preferred_element_type=jnp.float32) + bc1_ref[...]
        c = jnp.maximum(gc_ref[...] * (c * _BN_INV) + bec_ref[...], 0.0)
        c2 = jnp.maximum(
            jnp.dot(c, wc2_ref[...], preferred_element_type=jnp.float32) + bc2_ref[...],
            0.0,
        )
        ol_ref[...] = (
            jnp.dot(c2, wc3_ref[...], preferred_element_type=jnp.float32) + bc3_ref[...]
        )

    full = lambda shape: pl.BlockSpec(shape, lambda i: tuple(0 for _ in shape))
    return pl.pallas_call(
        body,
        grid=(_N // _ROWS,),
        in_specs=[
            pl.BlockSpec((_ROWS, _W), lambda i: (i, 0)),
            pl.BlockSpec((_ROWS, _W), lambda i: (i, 0)),
            pl.BlockSpec((_ROWS, _W), lambda i: (i, 0)),
            pl.BlockSpec((_ROWS, _H), lambda i: (i, 0)),
            full((3, _H, _H)),
            full((1, _H)),
            full((3, _H, _H)),
            full((3, 1, _H)),
            full((3, _H, _H)),
            full((3, 1, _H)),
            full((_H, _H)),
            full((1, _H)),
            full((1, _H)),
            full((1, _H)),
            full((_H, _H // 2)),
            full((1, _H // 2)),
            full((_H // 2, 8)),
            full((1, 8)),
        ],
        out_specs=[
            pl.BlockSpec((_ROWS, _H), lambda i: (i, 0)),
            pl.BlockSpec((_ROWS, 8), lambda i: (i, 0)),
        ],
        out_shape=[
            jax.ShapeDtypeStruct((_N, _H), jnp.float32),
            jax.ShapeDtypeStruct((_N, 8), jnp.float32),
        ],
    )(h0, h1, h2, hp, wjk_t, bjk, wih_t, bih, whh_t, bhh,
      wc1_t, bc1, gc, bec, wc2_t, bc2, wc3_t, bc3)


def kernel(x, edge_index, h_prev, W_in, b_in, Wl1, bl1, Wr1, Wl2, bl2, Wr2,
           g1, be1, g2, be2, Wjk, bjk, Wih, Whh, bih, bhh, Wc1, bc1, gc, bec,
           Wc2, bc2, Wc3, bc3):
    src = edge_index[0]
    dst = edge_index[1]

    w_in_t = W_in.T
    wl1_t, wr1_t = Wl1.T, Wr1.T
    wl2_t, wr2_t = Wl2.T, Wr2.T
    wjk_t = Wjk.T.reshape(3, _H, _H)
    wih_t = Wih.T.reshape(_H, 3, _H).transpose(1, 0, 2)
    whh_t = Whh.T.reshape(_H, 3, _H).transpose(1, 0, 2)
    bih3 = bih.reshape(3, 1, _H)
    bhh3 = bhh.reshape(3, 1, _H)
    wc1_t = Wc1.T
    wc2_t = Wc2.T
    wc3_t = jnp.zeros((_H // 2, 8), jnp.float32).at[:, :2].set(Wc3.T)
    bc3_p = jnp.zeros((1, 8), jnp.float32).at[:, :2].set(bc3.reshape(1, 2))
    row = lambda v: v.reshape(1, -1)

    h0 = _tc_input(x, w_in_t, row(b_in))
    agg1 = _sc_agg(h0, src, dst)
    h1 = _tc_sage(agg1, h0, wl1_t, row(bl1), wr1_t, row(g1), row(be1))
    agg2 = _sc_agg(h1, src, dst)
    h2 = _tc_sage(agg2, h1, wl2_t, row(bl2), wr2_t, row(g2), row(be2))
    h_curr, logits8 = _tc_head(
        h0, h1, h2, h_prev, wjk_t, row(bjk), wih_t, bih3, whh_t, bhh3,
        wc1_t, row(bc1), row(gc), row(bec), wc2_t, row(bc2), wc3_t, bc3_p)
    return (h_curr, logits8[:, :2])


# speedup vs baseline: 10.5078x; 2.2356x over previous
import functools
import math

import jax
import jax.numpy as jnp
from jax import lax
from jax.experimental import pallas as pl
from jax.experimental.pallas import tpu as pltpu
from jax.experimental.pallas import tpu_sc as plsc

_N = 10000
_E = 320000
_DF = 128
_H = 64
_W = 128

_NUM_SC = 2
_TILES = 16
_WORKERS = _NUM_SC * _TILES
_EW = _E // _WORKERS
_CHUNK = 50
_NBUF = 5
_NROUND = _EW // (_CHUNK * _NBUF)
_NPAD = 10240
_RPT = _NPAD // _TILES

_BN_INV = float(1.0 / math.sqrt(1.0 + 1e-5))


def _sc_agg(h, src4, dst4):
    @functools.partial(
        pl.kernel,
        out_type=jax.ShapeDtypeStruct((_NUM_SC, _NPAD, _W), jnp.float32),
        mesh=plsc.VectorSubcoreMesh(core_axis_name="c", subcore_axis_name="s"),
        scratch_types=(
            [pltpu.VMEM((2, _NBUF, _CHUNK), jnp.int32)] * 2
            + [pltpu.VMEM((_CHUNK, _W), jnp.float32)] * _NBUF
            + [pltpu.VMEM_SHARED((_NPAD, _W), jnp.float32)]
            + [pltpu.SemaphoreType.DMA] * (2 * _NBUF + 1)
        ),
    )
    def k(h_hbm, src_hbm, dst_hbm, out_hbm, sidx, didx, *rest):
        bufs = rest[:_NBUF]
        agg_s = rest[_NBUF]
        gsem = rest[_NBUF + 1 : 2 * _NBUF + 1]
        ssem = rest[2 * _NBUF + 1 : 3 * _NBUF + 1]
        isem = rest[3 * _NBUF + 1]
        cid = lax.axis_index("c")
        sid = lax.axis_index("s")
        wid = cid * _TILES + sid

        def idx_copies(r, p):
            return (
                pltpu.make_async_copy(src_hbm.at[wid, r], sidx.at[p], isem),
                pltpu.make_async_copy(dst_hbm.at[wid, r], didx.at[p], isem),
            )

        def gather(r_unused, p, b):
            return pltpu.make_async_copy(
                h_hbm.at[sidx.at[p, b]], bufs[b], gsem[b]
            )

        def scatter(r_unused, p, b):
            return pltpu.make_async_copy(
                bufs[b], agg_s.at[didx.at[p, b]], ssem[b]
            )

        @pl.loop(0, _CHUNK)
        def _(i):
            @pl.loop(0, _W // 16)
            def _(j):
                bufs[0][i, pl.ds(j * 16, 16)] = jnp.zeros((16,), jnp.float32)

        for c in idx_copies(0, 0):
            c.start()
        for c in idx_copies(0, 0):
            c.wait()

        @pl.loop(0, _RPT // _CHUNK)
        def _(kk):
            pltpu.sync_copy(bufs[0], agg_s.at[pl.ds(sid * _RPT + kk * _CHUNK, _CHUNK)])

        plsc.subcore_barrier()

        for c in idx_copies(1, 1):
            c.start()
        for b in range(_NBUF):
            gather(0, 0, b).start()

        @pl.loop(0, _NROUND - 1)
        def _(i):
            p = lax.rem(i, 2)
            pn = lax.rem(i + 1, 2)
            for b in range(_NBUF):
                gather(i, p, b).wait()
                pltpu.async_copy(bufs[b], agg_s.at[didx.at[p, b]], ssem[b], add=True)
            for c in idx_copies(i + 1, pn):
                c.wait()
            for b in range(_NBUF):
                scatter(i, p, b).wait()
                gather(i + 1, pn, b).start()

            @pl.when(i + 2 < _NROUND)
            def _():
                for c in idx_copies(i + 2, p):
                    c.start()

        pf = lax.rem(_NROUND - 1, 2)
        for b in range(_NBUF):
            gather(_NROUND - 1, pf, b).wait()
            pltpu.async_copy(bufs[b], agg_s.at[didx.at[pf, b]], ssem[b], add=True)
        for b in range(_NBUF):
            scatter(_NROUND - 1, pf, b).wait()

        plsc.subcore_barrier()
        pltpu.sync_copy(
            agg_s.at[pl.ds(sid * _RPT, _RPT)],
            out_hbm.at[cid, pl.ds(sid * _RPT, _RPT)],
        )

    return k(h, src4, dst4)


_ROWS = 2000


def _tag_tail(r):
    return (lax.broadcasted_iota(jnp.int32, (r, _W - _H), 1) == 0).astype(
        jnp.float32
    )


def _tc_input(x, w_t, b):
    def body(x_ref, w_ref, b_ref, o_ref):
        y = jnp.dot(x_ref[...], w_ref[...], preferred_element_type=jnp.float32)
        o_ref[:, : _H] = jnp.maximum(y + b_ref[...], 0.0)
        o_ref[:, _H :] = _tag_tail(_ROWS)

    return pl.pallas_call(
        body,
        grid=(_N // _ROWS,),
        in_specs=[
            pl.BlockSpec((_ROWS, _DF), lambda i: (i, 0)),
            pl.BlockSpec((_DF, _H), lambda i: (0, 0)),
            pl.BlockSpec((1, _H), lambda i: (0, 0)),
        ],
        out_specs=pl.BlockSpec((_ROWS, _W), lambda i: (i, 0)),
        out_shape=jax.ShapeDtypeStruct((_N, _W), jnp.float32),
    )(x, w_t, b)


def _tc_sage(aggp, h, wl_t, bl, wr_t, g, be):
    def body(a_ref, h_ref, wl_ref, bl_ref, wr_ref, g_ref, be_ref, o_ref):
        p = a_ref[0] + a_ref[1]
        cnt = p[:, _H : _H + 1]
        mean = p[:, : _H] / jnp.maximum(cnt, 1.0)
        hh = h_ref[:, : _H]
        y = (
            jnp.dot(mean, wl_ref[...], preferred_element_type=jnp.float32)
            + bl_ref[...]
            + jnp.dot(hh, wr_ref[...], preferred_element_type=jnp.float32)
        )
        y = g_ref[...] * (y * _BN_INV) + be_ref[...]
        o_ref[:, : _H] = jnp.maximum(y, 0.0) + hh
        o_ref[:, _H :] = _tag_tail(_ROWS)

    return pl.pallas_call(
        body,
        grid=(_N // _ROWS,),
        in_specs=[
            pl.BlockSpec((_NUM_SC, _ROWS, _W), lambda i: (0, i, 0)),
            pl.BlockSpec((_ROWS, _W), lambda i: (i, 0)),
            pl.BlockSpec((_H, _H), lambda i: (0, 0)),
            pl.BlockSpec((1, _H), lambda i: (0, 0)),
            pl.BlockSpec((_H, _H), lambda i: (0, 0)),
            pl.BlockSpec((1, _H), lambda i: (0, 0)),
            pl.BlockSpec((1, _H), lambda i: (0, 0)),
        ],
        out_specs=pl.BlockSpec((_ROWS, _W), lambda i: (i, 0)),
        out_shape=jax.ShapeDtypeStruct((_N, _W), jnp.float32),
    )(aggp, h, wl_t, bl, wr_t, g, be)


def _sigmoid(x):
    return 1.0 / (1.0 + jnp.exp(-x))


def _tc_head(h0, h1, h2, hp, wjk_t, bjk, wih_t, bih, whh_t, bhh,
             wc1_t, bc1, gc, bec, wc2_t, bc2, wc3_t, bc3):
    def body(h0_ref, h1_ref, h2_ref, hp_ref, wjk_ref, bjk_ref, wih_ref, bih_ref,
             whh_ref, bhh_ref, wc1_ref, bc1_ref, gc_ref, bec_ref, wc2_ref,
             bc2_ref, wc3_ref, bc3_ref, oh_ref, ol_ref):
        hj = (
            jnp.dot(h0_ref[:, : _H], wjk_ref[0], preferred_element_type=jnp.float32)
            + jnp.dot(h1_ref[:, : _H], wjk_ref[1], preferred_element_type=jnp.float32)
            + jnp.dot(h2_ref[:, : _H], wjk_ref[2], preferred_element_type=jnp.float32)
        )
        h = jnp.maximum(hj + bjk_ref[...], 0.0)
        hprev = hp_ref[...]
        ir = jnp.dot(h, wih_ref[0], preferred_element_type=jnp.float32) + bih_ref[0]
        iz = jnp.dot(h, wih_ref[1], preferred_element_type=jnp.float32) + bih_ref[1]
        inn = jnp.dot(h, wih_ref[2], preferred_element_type=jnp.float32) + bih_ref[2]
        hr = jnp.dot(hprev, whh_ref[0], preferred_element_type=jnp.float32) + bhh_ref[0]
        hz = jnp.dot(hprev, whh_ref[1], preferred_element_type=jnp.float32) + bhh_ref[1]
        hn = jnp.dot(hprev, whh_ref[2], preferred_element_type=jnp.float32) + bhh_ref[2]
        r = _sigmoid(ir + hr)
        z = _sigmoid(iz + hz)
        n = jnp.tanh(inn + r * hn)
        hc = (1.0 - z) * n + z * hprev
        oh_ref[...] = hc
        c = jnp.dot(hc, wc1_ref[...], preferred_element_type=jnp.float32) + bc1_ref[...]
        c = jnp.maximum(gc_ref[...] * (c * _BN_INV) + bec_ref[...], 0.0)
        c2 = jnp.maximum(
            jnp.dot(c, wc2_ref[...], preferred_element_type=jnp.float32) + bc2_ref[...],
            0.0,
        )
        ol_ref[...] = (
            jnp.dot(c2, wc3_ref[...], preferred_element_type=jnp.float32) + bc3_ref[...]
        )

    full = lambda shape: pl.BlockSpec(shape, lambda i: tuple(0 for _ in shape))
    return pl.pallas_call(
        body,
        grid=(_N // _ROWS,),
        in_specs=[
            pl.BlockSpec((_ROWS, _W), lambda i: (i, 0)),
            pl.BlockSpec((_ROWS, _W), lambda i: (i, 0)),
            pl.BlockSpec((_ROWS, _W), lambda i: (i, 0)),
            pl.BlockSpec((_ROWS, _H), lambda i: (i, 0)),
            full((3, _H, _H)),
            full((1, _H)),
            full((3, _H, _H)),
            full((3, 1, _H)),
            full((3, _H, _H)),
            full((3, 1, _H)),
            full((_H, _H)),
            full((1, _H)),
            full((1, _H)),
            full((1, _H)),
            full((_H, _H // 2)),
            full((1, _H // 2)),
            full((_H // 2, 8)),
            full((1, 8)),
        ],
        out_specs=[
            pl.BlockSpec((_ROWS, _H), lambda i: (i, 0)),
            pl.BlockSpec((_ROWS, 8), lambda i: (i, 0)),
        ],
        out_shape=[
            jax.ShapeDtypeStruct((_N, _H), jnp.float32),
            jax.ShapeDtypeStruct((_N, 8), jnp.float32),
        ],
    )(h0, h1, h2, hp, wjk_t, bjk, wih_t, bih, whh_t, bhh,
      wc1_t, bc1, gc, bec, wc2_t, bc2, wc3_t, bc3)


def kernel(x, edge_index, h_prev, W_in, b_in, Wl1, bl1, Wr1, Wl2, bl2, Wr2,
           g1, be1, g2, be2, Wjk, bjk, Wih, Whh, bih, bhh, Wc1, bc1, gc, bec,
           Wc2, bc2, Wc3, bc3):
    src4 = edge_index[0].reshape(_WORKERS, _NROUND, _NBUF, _CHUNK)
    dst4 = edge_index[1].reshape(_WORKERS, _NROUND, _NBUF, _CHUNK)

    w_in_t = W_in.T
    wl1_t, wr1_t = Wl1.T, Wr1.T
    wl2_t, wr2_t = Wl2.T, Wr2.T
    wjk_t = Wjk.T.reshape(3, _H, _H)
    wih_t = Wih.T.reshape(_H, 3, _H).transpose(1, 0, 2)
    whh_t = Whh.T.reshape(_H, 3, _H).transpose(1, 0, 2)
    bih3 = bih.reshape(3, 1, _H)
    bhh3 = bhh.reshape(3, 1, _H)
    wc1_t = Wc1.T
    wc2_t = Wc2.T
    wc3_t = jnp.zeros((_H // 2, 8), jnp.float32).at[:, :2].set(Wc3.T)
    bc3_p = jnp.zeros((1, 8), jnp.float32).at[:, :2].set(bc3.reshape(1, 2))
    row = lambda v: v.reshape(1, -1)

    h0 = _tc_input(x, w_in_t, row(b_in))
    agg1 = _sc_agg(h0, src4, dst4)
    h1 = _tc_sage(agg1, h0, wl1_t, row(bl1), wr1_t, row(g1), row(be1))
    agg2 = _sc_agg(h1, src4, dst4)
    h2 = _tc_sage(agg2, h1, wl2_t, row(bl2), wr2_t, row(g2), row(be2))
    h_curr, logits8 = _tc_head(
        h0, h1, h2, h_prev, wjk_t, row(bjk), wih_t, bih3, whh_t, bhh3,
        wc1_t, row(bc1), row(gc), row(bec), wc2_t, row(bc2), wc3_t, bc3_p)
    return (h_curr, logits8[:, :2])
